# Initial kernel scaffold; baseline (speedup 1.0000x reference)
import jax
import jax.numpy as jnp
from jax.experimental import pallas as pl


def kernel(place_x, transition_x, edge_index_pre, edge_index_post, Wp, bp, Wt, bt, Wl_pre_0, bl_pre_0, Wr_pre_0, Wl_post_0, bl_post_0, Wr_post_0, Wl_pre_1, bl_pre_1, Wr_pre_1, Wl_post_1, bl_post_1, Wr_post_1, Wc1, bc1, Wc2, bc2):
    raise NotImplementedError("write your pallas kernel here")


# speedup vs baseline: 4.9159x; 4.9159x over previous
import functools

import jax
import jax.numpy as jnp
from jax import lax
from jax.experimental import pallas as pl
from jax.experimental.pallas import tpu as pltpu
from jax.experimental.pallas import tpu_sc as plsc

N = 50000
H = 64
E = 800000
NC = 2
NS = 16
G = 128
E_PAD = 802816
TRASH = N
ACC_ROWS = 50048
CHUNK = ACC_ROWS // NS
EW1 = E_PAD // (NC * NS)
G1 = EW1 // G
EW3 = E_PAD // NS
G3 = EW3 // G
BLK = 1000
NBLK = N // BLK

_mesh = plsc.VectorSubcoreMesh(core_axis_name="c", subcore_axis_name="s")


@functools.partial(
    pl.kernel,
    out_type=[jax.ShapeDtypeStruct((NC, ACC_ROWS, 16), jnp.float32),
              jax.ShapeDtypeStruct((NC, ACC_ROWS, 16), jnp.float32)],
    mesh=_mesh,
    scratch_types=[
        pltpu.VMEM((1, G), jnp.int32),
        pltpu.VMEM((1, G), jnp.int32),
        pltpu.VMEM((G, 16), jnp.float32),
        pltpu.VMEM_SHARED((ACC_ROWS, 16), jnp.float32),
    ],
    compiler_params=pltpu.CompilerParams(use_tc_tiling_on_sc=False),
)
def _phase1(tab_pre, tab_post, src_pre, dst_pre, src_post, dst_post, zz,
            out_pre, out_post, sidx, didx, vals, acc):
    c = lax.axis_index("c")
    s = lax.axis_index("s")
    wid = c * NS + s
    for tab, srcv, dstv, out in ((tab_pre, src_pre, dst_pre, out_pre),
                                 (tab_post, src_post, dst_post, out_post)):
        pltpu.sync_copy(zz.at[pl.ds(s * CHUNK, CHUNK)],
                        acc.at[pl.ds(s * CHUNK, CHUNK)])
        plsc.subcore_barrier()

        def body(g, carry, srcv=srcv, dstv=dstv, tab=tab):
            off = wid * EW1 + g * G
            pltpu.sync_copy(srcv.at[pl.ds(off, G)], sidx.at[0])
            pltpu.sync_copy(dstv.at[pl.ds(off, G)], didx.at[0])
            pltpu.sync_copy(tab.at[sidx.at[0]], vals)
            pltpu.sync_copy(vals, acc.at[didx.at[0]], add=True)
            return carry

        lax.fori_loop(0, G1, body, 0)
        plsc.subcore_barrier()
        pltpu.sync_copy(acc.at[pl.ds(s * CHUNK, CHUNK)],
                        out.at[c, pl.ds(s * CHUNK, CHUNK)])
        plsc.subcore_barrier()


@functools.partial(
    pl.kernel,
    out_type=[jax.ShapeDtypeStruct((NC, ACC_ROWS, 32), jnp.float32),
              jax.ShapeDtypeStruct((NC, ACC_ROWS, 32), jnp.float32)],
    mesh=_mesh,
    scratch_types=[
        pltpu.VMEM((1, G), jnp.int32),
        pltpu.VMEM((1, G), jnp.int32),
        pltpu.VMEM((1, G), jnp.int32),
        pltpu.VMEM((G, 32), jnp.float32),
        pltpu.VMEM_SHARED((ACC_ROWS, 32), jnp.float32),
    ],
    compiler_params=pltpu.CompilerParams(use_tc_tiling_on_sc=False),
)
def _phase3(p1t, t1t, src_pre, dst_pre, src_post, dst_post, zz,
            out_pre, out_post, sidx, lidx, didx, rows, acc):
    c = lax.axis_index("c")
    s = lax.axis_index("s")
    base = c * N
    for tab, srcv, dstv, out in ((p1t, src_pre, dst_pre, out_pre),
                                 (t1t, src_post, dst_post, out_post)):
        pltpu.sync_copy(zz.at[pl.ds(s * CHUNK, CHUNK)],
                        acc.at[pl.ds(s * CHUNK, CHUNK)])
        plsc.subcore_barrier()

        def body(g, carry, srcv=srcv, dstv=dstv, tab=tab):
            off = s * EW3 + g * G
            pltpu.sync_copy(srcv.at[pl.ds(off, G)], sidx.at[0])
            pltpu.sync_copy(dstv.at[pl.ds(off, G)], didx.at[0])
            for j in range(G // 16):
                lidx[0, pl.ds(j * 16, 16)] = sidx[0, pl.ds(j * 16, 16)] + base
            pltpu.sync_copy(tab.at[lidx.at[0]], rows)
            pltpu.sync_copy(rows, acc.at[didx.at[0]], add=True)
            return carry

        lax.fori_loop(0, G3, body, 0)
        plsc.subcore_barrier()
        pltpu.sync_copy(acc.at[pl.ds(s * CHUNK, CHUNK)],
                        out.at[c, pl.ds(s * CHUNK, CHUNK)])
        plsc.subcore_barrier()


def _phase2_body(accpre_ref, accpost_ref, px_ref, tx_ref, Wp_ref, bp_ref,
                 Wt_ref, bt_ref, Wlp0_ref, blp0_ref, Wrp0_ref, Wlq0_ref,
                 blq0_ref, Wrq0_ref, t1o_ref, p1o_ref):
    f32 = jnp.float32
    accpre = accpre_ref[...]
    accpost = accpost_ref[...]
    spre = accpre[0, :, 0:1] + accpre[1, :, 0:1]
    cpre = accpre[0, :, 1:2] + accpre[1, :, 1:2]
    spost = accpost[0, :, 0:1] + accpost[1, :, 0:1]
    cpost = accpost[0, :, 1:2] + accpost[1, :, 1:2]
    upre = spre / jnp.maximum(cpre, 1.0)
    vpre = (cpre > 0).astype(f32)
    upost = spost / jnp.maximum(cpost, 1.0)
    vpost = (cpost > 0).astype(f32)
    Wp0 = Wp_ref[...]
    Wt0 = Wt_ref[...]
    Wlp0 = Wlp0_ref[...]
    Wrp0 = Wrp0_ref[...]
    Wlq0 = Wlq0_ref[...]
    Wrq0 = Wrq0_ref[...]
    bp = bp_ref[...]
    bt = bt_ref[...]
    dot = lambda a, b: jnp.dot(a, b, preferred_element_type=f32,
                               precision=jax.lax.Precision.HIGHEST)
    A1 = dot(Wp0, Wlp0)
    A2 = dot(bp, Wlp0)
    A3 = dot(Wt0, Wrp0)
    A4 = blp0_ref[...] + dot(bt, Wrp0)
    B1 = dot(Wt0, Wlq0)
    B2 = dot(bt, Wlq0)
    B3 = dot(Wp0, Wrq0)
    B4 = blq0_ref[...] + dot(bp, Wrq0)
    tx = tx_ref[...]
    px = px_ref[...]
    t1 = jnp.maximum(upre * A1 + vpre * A2 + tx * A3 + A4, 0.0)
    p1 = jnp.maximum(upost * B1 + vpost * B2 + px * B3 + B4, 0.0)
    t1o_ref[0] = t1[:, :32]
    t1o_ref[1] = t1[:, 32:]
    p1o_ref[0] = p1[:, :32]
    p1o_ref[1] = p1[:, 32:]


def _phase2(accpre, accpost, px, tx, Wp, bp, Wt, bt,
            Wlp0, blp0, Wrp0, Wlq0, blq0, Wrq0):
    full = lambda shape: pl.BlockSpec(shape, lambda i: tuple(0 for _ in shape))
    return pl.pallas_call(
        _phase2_body,
        grid=(NBLK,),
        in_specs=[
            pl.BlockSpec((2, BLK, 2), lambda i: (0, i, 0)),
            pl.BlockSpec((2, BLK, 2), lambda i: (0, i, 0)),
            pl.BlockSpec((BLK, 1), lambda i: (i, 0)),
            pl.BlockSpec((BLK, 1), lambda i: (i, 0)),
            full((1, H)), full((1, H)), full((1, H)), full((1, H)),
            full((H, H)), full((1, H)), full((H, H)),
            full((H, H)), full((1, H)), full((H, H)),
        ],
        out_specs=[
            pl.BlockSpec((2, BLK, 32), lambda i: (0, i, 0)),
            pl.BlockSpec((2, BLK, 32), lambda i: (0, i, 0)),
        ],
        out_shape=[jax.ShapeDtypeStruct((2, N, 32), jnp.float32),
                   jax.ShapeDtypeStruct((2, N, 32), jnp.float32)],
        compiler_params=pltpu.CompilerParams(
            dimension_semantics=("arbitrary",)),
    )(accpre, accpost, px, tx, Wp, bp, Wt, bt,
      Wlp0, blp0, Wrp0, Wlq0, blq0, Wrq0)


def _phase4_body(Spre_ref, Spost_ref, accpre_ref, accpost_ref, t1o_ref,
                 p1o_ref, Wlp1_ref, blp1_ref, Wrp1_ref, Wlq1_ref, blq1_ref,
                 Wrq1_ref, Wc1_ref, bc1_ref, Wc2_ref, bc2_ref, out_ref,
                 accP, accT):
    i = pl.program_id(0)
    f32 = jnp.float32
    dot = lambda a, b: jnp.dot(a, b, preferred_element_type=f32,
                               precision=jax.lax.Precision.HIGHEST)
    accpre = accpre_ref[...]
    accpost = accpost_ref[...]
    cpre = accpre[0, :, 1:2] + accpre[1, :, 1:2]
    cpost = accpost[0, :, 1:2] + accpost[1, :, 1:2]
    Spre = jnp.concatenate([Spre_ref[0], Spre_ref[1]], axis=1)
    Spost = jnp.concatenate([Spost_ref[0], Spost_ref[1]], axis=1)
    meanpre = Spre / jnp.maximum(cpre, 1.0)
    meanpost = Spost / jnp.maximum(cpost, 1.0)
    t1 = jnp.concatenate([t1o_ref[0], t1o_ref[1]], axis=1)
    p1 = jnp.concatenate([p1o_ref[0], p1o_ref[1]], axis=1)
    t2 = jnp.maximum(
        dot(meanpre, Wlp1_ref[...]) + blp1_ref[...] + dot(t1, Wrp1_ref[...]),
        0.0)
    p2 = jnp.maximum(
        dot(meanpost, Wlq1_ref[...]) + blq1_ref[...] + dot(p1, Wrq1_ref[...]),
        0.0)
    pt = jnp.sum(p2, axis=0, keepdims=True)
    tt = jnp.sum(t2, axis=0, keepdims=True)

    @pl.when(i == 0)
    def _():
        accP[...] = pt
        accT[...] = tt

    @pl.when(i > 0)
    def _():
        accP[...] += pt
        accT[...] += tt

    @pl.when(i == NBLK - 1)
    def _():
        mp = accP[...] / f32(N)
        mt = accT[...] / f32(N)
        g = jnp.concatenate([mp, mt], axis=1)
        h = jnp.maximum(dot(g, Wc1_ref[...]) + bc1_ref[...], 0.0)
        out_ref[...] = dot(h, Wc2_ref[...]) + bc2_ref[...]


def _phase4(Spre, Spost, accpre, accpost, t1o, p1o,
            Wlp1, blp1, Wrp1, Wlq1, blq1, Wrq1, Wc1, bc1, Wc2, bc2):
    full = lambda shape: pl.BlockSpec(shape, lambda i: tuple(0 for _ in shape))
    return pl.pallas_call(
        _phase4_body,
        grid=(NBLK,),
        in_specs=[
            pl.BlockSpec((2, BLK, 32), lambda i: (0, i, 0)),
            pl.BlockSpec((2, BLK, 32), lambda i: (0, i, 0)),
            pl.BlockSpec((2, BLK, 2), lambda i: (0, i, 0)),
            pl.BlockSpec((2, BLK, 2), lambda i: (0, i, 0)),
            pl.BlockSpec((2, BLK, 32), lambda i: (0, i, 0)),
            pl.BlockSpec((2, BLK, 32), lambda i: (0, i, 0)),
            full((H, H)), full((1, H)), full((H, H)),
            full((H, H)), full((1, H)), full((H, H)),
            full((2 * H, H)), full((1, H)), full((H, 2)), full((1, 2)),
        ],
        out_specs=[pl.BlockSpec((1, 2), lambda i: (0, 0))],
        out_shape=[jax.ShapeDtypeStruct((1, 2), jnp.float32)],
        scratch_shapes=[pltpu.VMEM((1, H), jnp.float32),
                        pltpu.VMEM((1, H), jnp.float32)],
        compiler_params=pltpu.CompilerParams(
            dimension_semantics=("arbitrary",)),
    )(Spre, Spost, accpre, accpost, t1o, p1o,
      Wlp1, blp1, Wrp1, Wlq1, blq1, Wrq1, Wc1, bc1, Wc2, bc2)[0]


def kernel(place_x, transition_x, edge_index_pre, edge_index_post, Wp, bp,
           Wt, bt, Wl_pre_0, bl_pre_0, Wr_pre_0, Wl_post_0, bl_post_0,
           Wr_post_0, Wl_pre_1, bl_pre_1, Wr_pre_1, Wl_post_1, bl_post_1,
           Wr_post_1, Wc1, bc1, Wc2, bc2):
    f32 = jnp.float32
    i32 = jnp.int32
    ones = jnp.ones_like(place_x)
    zpad = jnp.zeros((N, 14), f32)
    tab_pre = jnp.concatenate([place_x, ones, zpad], axis=1)
    tab_post = jnp.concatenate([transition_x, ones, zpad], axis=1)
    pad = E_PAD - E
    pad_src = jnp.zeros((pad,), i32)
    pad_dst = jnp.full((pad,), TRASH, i32)
    src_pre = jnp.concatenate([edge_index_pre[0].astype(i32), pad_src])
    dst_pre = jnp.concatenate([edge_index_pre[1].astype(i32), pad_dst])
    src_post = jnp.concatenate([edge_index_post[0].astype(i32), pad_src])
    dst_post = jnp.concatenate([edge_index_post[1].astype(i32), pad_dst])
    zz2 = jnp.zeros((ACC_ROWS, 16), f32)
    zz32 = jnp.zeros((ACC_ROWS, 32), f32)

    acc_pre, acc_post = _phase1(tab_pre, tab_post, src_pre, dst_pre,
                                src_post, dst_post, zz2)
    acc_pre = acc_pre[:, :N, 0:2]
    acc_post = acc_post[:, :N, 0:2]

    r = lambda b: b.reshape(1, -1)
    t1o, p1o = _phase2(acc_pre, acc_post, place_x, transition_x,
                       Wp, r(bp), Wt, r(bt),
                       Wl_pre_0, r(bl_pre_0), Wr_pre_0,
                       Wl_post_0, r(bl_post_0), Wr_post_0)

    Spre, Spost = _phase3(p1o.reshape(2 * N, 32), t1o.reshape(2 * N, 32),
                          src_pre, dst_pre, src_post, dst_post, zz32)
    Spre = Spre[:, :N, :]
    Spost = Spost[:, :N, :]

    return _phase4(Spre, Spost, acc_pre, acc_post, t1o, p1o,
                   Wl_pre_1, r(bl_pre_1), Wr_pre_1,
                   Wl_post_1, r(bl_post_1), Wr_post_1,
                   Wc1, r(bc1), Wc2, r(bc2))


# speedup vs baseline: 9.9892x; 2.0320x over previous
import functools

import jax
import jax.numpy as jnp
from jax import lax
from jax.experimental import pallas as pl
from jax.experimental.pallas import tpu as pltpu
from jax.experimental.pallas import tpu_sc as plsc

N = 50000
H = 64
E = 800000
NC = 2
NS = 16
G = 128
E_PAD = 802816
TRASH = N
ACC_ROWS = 50048
CHUNK = ACC_ROWS // NS
ROWS_E = E_PAD // G
G1 = ROWS_E // (NC * NS)
SEG1 = 28
CH1 = 7
G3 = ROWS_E // NS
SEG3 = 28
CH3 = 4
BLK = 1000
NBLK = N // BLK

_mesh = plsc.VectorSubcoreMesh(core_axis_name="c", subcore_axis_name="s")


@functools.partial(
    pl.kernel,
    out_type=[jax.ShapeDtypeStruct((NC, ACC_ROWS, 16), jnp.float32),
              jax.ShapeDtypeStruct((NC, ACC_ROWS, 16), jnp.float32)],
    mesh=_mesh,
    scratch_types=[
        pltpu.VMEM((SEG1, G), jnp.int32),
        pltpu.VMEM((SEG1, G), jnp.int32),
        pltpu.VMEM((CH1, G, 16), jnp.float32),
        pltpu.VMEM_SHARED((ACC_ROWS, 16), jnp.float32),
        pltpu.SemaphoreType.DMA,
        pltpu.SemaphoreType.DMA,
    ],
    compiler_params=pltpu.CompilerParams(use_tc_tiling_on_sc=False),
)
def _phase1(tab_pre, tab_post, src_pre, dst_pre, src_post, dst_post, zz,
            out_pre, out_post, sbuf, dbuf, rows, acc, gsem, ssem):
    c = lax.axis_index("c")
    s = lax.axis_index("s")
    wid = c * NS + s
    for tab, srcv, dstv, out in ((tab_pre, src_pre, dst_pre, out_pre),
                                 (tab_post, src_post, dst_post, out_post)):
        pltpu.sync_copy(zz.at[pl.ds(s * CHUNK, CHUNK)],
                        acc.at[pl.ds(s * CHUNK, CHUNK)])
        plsc.subcore_barrier()

        def seg_body(seg, carry, srcv=srcv, dstv=dstv, tab=tab):
            row0 = wid * G1 + seg * SEG1
            pltpu.sync_copy(srcv.at[pl.ds(row0, SEG1)], sbuf)
            pltpu.sync_copy(dstv.at[pl.ds(row0, SEG1)], dbuf)

            def ch_body(ck, carry2, tab=tab):
                gh = []
                for b in range(CH1):
                    r = ck * CH1 + b
                    gh.append(pltpu.async_copy(tab.at[sbuf.at[r]],
                                               rows.at[b], gsem))
                for h in gh:
                    h.wait()
                sh = []
                for b in range(CH1):
                    r = ck * CH1 + b
                    sh.append(pltpu.async_copy(rows.at[b], acc.at[dbuf.at[r]],
                                               ssem, add=True))
                for h in sh:
                    h.wait()
                return carry2

            lax.fori_loop(0, SEG1 // CH1, ch_body, 0)
            return carry

        lax.fori_loop(0, G1 // SEG1, seg_body, 0)
        plsc.subcore_barrier()
        pltpu.sync_copy(acc.at[pl.ds(s * CHUNK, CHUNK)],
                        out.at[c, pl.ds(s * CHUNK, CHUNK)])
        plsc.subcore_barrier()


@functools.partial(
    pl.kernel,
    out_type=[jax.ShapeDtypeStruct((NC, ACC_ROWS, 32), jnp.float32),
              jax.ShapeDtypeStruct((NC, ACC_ROWS, 32), jnp.float32)],
    mesh=_mesh,
    scratch_types=[
        pltpu.VMEM((SEG3, G), jnp.int32),
        pltpu.VMEM((SEG3, G), jnp.int32),
        pltpu.VMEM((CH3, G), jnp.int32),
        pltpu.VMEM((CH3, G, 32), jnp.float32),
        pltpu.VMEM_SHARED((ACC_ROWS, 32), jnp.float32),
        pltpu.SemaphoreType.DMA,
        pltpu.SemaphoreType.DMA,
    ],
    compiler_params=pltpu.CompilerParams(use_tc_tiling_on_sc=False),
)
def _phase3(p1t, t1t, src_pre, dst_pre, src_post, dst_post, zz,
            out_pre, out_post, sbuf, dbuf, lbuf, rows, acc, gsem, ssem):
    c = lax.axis_index("c")
    s = lax.axis_index("s")
    base = c * N
    for tab, srcv, dstv, out in ((p1t, src_pre, dst_pre, out_pre),
                                 (t1t, src_post, dst_post, out_post)):
        pltpu.sync_copy(zz.at[pl.ds(s * CHUNK, CHUNK)],
                        acc.at[pl.ds(s * CHUNK, CHUNK)])
        plsc.subcore_barrier()

        def seg_body(seg, carry, srcv=srcv, dstv=dstv, tab=tab):
            row0 = s * G3 + seg * SEG3
            pltpu.sync_copy(srcv.at[pl.ds(row0, SEG3)], sbuf)
            pltpu.sync_copy(dstv.at[pl.ds(row0, SEG3)], dbuf)

            def ch_body(ck, carry2, tab=tab):
                gh = []
                for b in range(CH3):
                    r = ck * CH3 + b
                    for j in range(G // 16):
                        lbuf[b, pl.ds(j * 16, 16)] = (
                            sbuf[r, pl.ds(j * 16, 16)] + base)
                    gh.append(pltpu.async_copy(tab.at[lbuf.at[b]],
                                               rows.at[b], gsem))
                for h in gh:
                    h.wait()
                sh = []
                for b in range(CH3):
                    r = ck * CH3 + b
                    sh.append(pltpu.async_copy(rows.at[b], acc.at[dbuf.at[r]],
                                               ssem, add=True))
                for h in sh:
                    h.wait()
                return carry2

            lax.fori_loop(0, SEG3 // CH3, ch_body, 0)
            return carry

        lax.fori_loop(0, G3 // SEG3, seg_body, 0)
        plsc.subcore_barrier()
        pltpu.sync_copy(acc.at[pl.ds(s * CHUNK, CHUNK)],
                        out.at[c, pl.ds(s * CHUNK, CHUNK)])
        plsc.subcore_barrier()


def _phase2_body(accpre_ref, accpost_ref, px_ref, tx_ref, Wp_ref, bp_ref,
                 Wt_ref, bt_ref, Wlp0_ref, blp0_ref, Wrp0_ref, Wlq0_ref,
                 blq0_ref, Wrq0_ref, t1o_ref, p1o_ref):
    f32 = jnp.float32
    accpre = accpre_ref[...]
    accpost = accpost_ref[...]
    spre = accpre[0, :, 0:1] + accpre[1, :, 0:1]
    cpre = accpre[0, :, 1:2] + accpre[1, :, 1:2]
    spost = accpost[0, :, 0:1] + accpost[1, :, 0:1]
    cpost = accpost[0, :, 1:2] + accpost[1, :, 1:2]
    upre = spre / jnp.maximum(cpre, 1.0)
    vpre = (cpre > 0).astype(f32)
    upost = spost / jnp.maximum(cpost, 1.0)
    vpost = (cpost > 0).astype(f32)
    Wp0 = Wp_ref[...]
    Wt0 = Wt_ref[...]
    Wlp0 = Wlp0_ref[...]
    Wrp0 = Wrp0_ref[...]
    Wlq0 = Wlq0_ref[...]
    Wrq0 = Wrq0_ref[...]
    bp = bp_ref[...]
    bt = bt_ref[...]
    dot = lambda a, b: jnp.dot(a, b, preferred_element_type=f32,
                               precision=jax.lax.Precision.HIGHEST)
    A1 = dot(Wp0, Wlp0)
    A2 = dot(bp, Wlp0)
    A3 = dot(Wt0, Wrp0)
    A4 = blp0_ref[...] + dot(bt, Wrp0)
    B1 = dot(Wt0, Wlq0)
    B2 = dot(bt, Wlq0)
    B3 = dot(Wp0, Wrq0)
    B4 = blq0_ref[...] + dot(bp, Wrq0)
    tx = tx_ref[...]
    px = px_ref[...]
    t1 = jnp.maximum(upre * A1 + vpre * A2 + tx * A3 + A4, 0.0)
    p1 = jnp.maximum(upost * B1 + vpost * B2 + px * B3 + B4, 0.0)
    t1o_ref[0] = t1[:, :32]
    t1o_ref[1] = t1[:, 32:]
    p1o_ref[0] = p1[:, :32]
    p1o_ref[1] = p1[:, 32:]


def _phase2(accpre, accpost, px, tx, Wp, bp, Wt, bt,
            Wlp0, blp0, Wrp0, Wlq0, blq0, Wrq0):
    full = lambda shape: pl.BlockSpec(shape, lambda i: tuple(0 for _ in shape))
    return pl.pallas_call(
        _phase2_body,
        grid=(NBLK,),
        in_specs=[
            pl.BlockSpec((2, BLK, 2), lambda i: (0, i, 0)),
            pl.BlockSpec((2, BLK, 2), lambda i: (0, i, 0)),
            pl.BlockSpec((BLK, 1), lambda i: (i, 0)),
            pl.BlockSpec((BLK, 1), lambda i: (i, 0)),
            full((1, H)), full((1, H)), full((1, H)), full((1, H)),
            full((H, H)), full((1, H)), full((H, H)),
            full((H, H)), full((1, H)), full((H, H)),
        ],
        out_specs=[
            pl.BlockSpec((2, BLK, 32), lambda i: (0, i, 0)),
            pl.BlockSpec((2, BLK, 32), lambda i: (0, i, 0)),
        ],
        out_shape=[jax.ShapeDtypeStruct((2, N, 32), jnp.float32),
                   jax.ShapeDtypeStruct((2, N, 32), jnp.float32)],
        compiler_params=pltpu.CompilerParams(
            dimension_semantics=("arbitrary",)),
    )(accpre, accpost, px, tx, Wp, bp, Wt, bt,
      Wlp0, blp0, Wrp0, Wlq0, blq0, Wrq0)


def _phase4_body(Spre_ref, Spost_ref, accpre_ref, accpost_ref, t1o_ref,
                 p1o_ref, Wlp1_ref, blp1_ref, Wrp1_ref, Wlq1_ref, blq1_ref,
                 Wrq1_ref, Wc1_ref, bc1_ref, Wc2_ref, bc2_ref, out_ref,
                 accP, accT):
    i = pl.program_id(0)
    f32 = jnp.float32
    dot = lambda a, b: jnp.dot(a, b, preferred_element_type=f32,
                               precision=jax.lax.Precision.HIGHEST)
    accpre = accpre_ref[...]
    accpost = accpost_ref[...]
    cpre = accpre[0, :, 1:2] + accpre[1, :, 1:2]
    cpost = accpost[0, :, 1:2] + accpost[1, :, 1:2]
    Spre = jnp.concatenate([Spre_ref[0], Spre_ref[1]], axis=1)
    Spost = jnp.concatenate([Spost_ref[0], Spost_ref[1]], axis=1)
    meanpre = Spre / jnp.maximum(cpre, 1.0)
    meanpost = Spost / jnp.maximum(cpost, 1.0)
    t1 = jnp.concatenate([t1o_ref[0], t1o_ref[1]], axis=1)
    p1 = jnp.concatenate([p1o_ref[0], p1o_ref[1]], axis=1)
    t2 = jnp.maximum(
        dot(meanpre, Wlp1_ref[...]) + blp1_ref[...] + dot(t1, Wrp1_ref[...]),
        0.0)
    p2 = jnp.maximum(
        dot(meanpost, Wlq1_ref[...]) + blq1_ref[...] + dot(p1, Wrq1_ref[...]),
        0.0)
    pt = jnp.sum(p2, axis=0, keepdims=True)
    tt = jnp.sum(t2, axis=0, keepdims=True)

    @pl.when(i == 0)
    def _():
        accP[...] = pt
        accT[...] = tt

    @pl.when(i > 0)
    def _():
        accP[...] += pt
        accT[...] += tt

    @pl.when(i == NBLK - 1)
    def _():
        mp = accP[...] / f32(N)
        mt = accT[...] / f32(N)
        g = jnp.concatenate([mp, mt], axis=1)
        h = jnp.maximum(dot(g, Wc1_ref[...]) + bc1_ref[...], 0.0)
        out_ref[...] = dot(h, Wc2_ref[...]) + bc2_ref[...]


def _phase4(Spre, Spost, accpre, accpost, t1o, p1o,
            Wlp1, blp1, Wrp1, Wlq1, blq1, Wrq1, Wc1, bc1, Wc2, bc2):
    full = lambda shape: pl.BlockSpec(shape, lambda i: tuple(0 for _ in shape))
    return pl.pallas_call(
        _phase4_body,
        grid=(NBLK,),
        in_specs=[
            pl.BlockSpec((2, BLK, 32), lambda i: (0, i, 0)),
            pl.BlockSpec((2, BLK, 32), lambda i: (0, i, 0)),
            pl.BlockSpec((2, BLK, 2), lambda i: (0, i, 0)),
            pl.BlockSpec((2, BLK, 2), lambda i: (0, i, 0)),
            pl.BlockSpec((2, BLK, 32), lambda i: (0, i, 0)),
            pl.BlockSpec((2, BLK, 32), lambda i: (0, i, 0)),
            full((H, H)), full((1, H)), full((H, H)),
            full((H, H)), full((1, H)), full((H, H)),
            full((2 * H, H)), full((1, H)), full((H, 2)), full((1, 2)),
        ],
        out_specs=[pl.BlockSpec((1, 2), lambda i: (0, 0))],
        out_shape=[jax.ShapeDtypeStruct((1, 2), jnp.float32)],
        scratch_shapes=[pltpu.VMEM((1, H), jnp.float32),
                        pltpu.VMEM((1, H), jnp.float32)],
        compiler_params=pltpu.CompilerParams(
            dimension_semantics=("arbitrary",)),
    )(Spre, Spost, accpre, accpost, t1o, p1o,
      Wlp1, blp1, Wrp1, Wlq1, blq1, Wrq1, Wc1, bc1, Wc2, bc2)[0]


def kernel(place_x, transition_x, edge_index_pre, edge_index_post, Wp, bp,
           Wt, bt, Wl_pre_0, bl_pre_0, Wr_pre_0, Wl_post_0, bl_post_0,
           Wr_post_0, Wl_pre_1, bl_pre_1, Wr_pre_1, Wl_post_1, bl_post_1,
           Wr_post_1, Wc1, bc1, Wc2, bc2):
    f32 = jnp.float32
    i32 = jnp.int32
    ones = jnp.ones_like(place_x)
    zpad = jnp.zeros((N, 14), f32)
    tab_pre = jnp.concatenate([place_x, ones, zpad], axis=1)
    tab_post = jnp.concatenate([transition_x, ones, zpad], axis=1)
    pad = E_PAD - E
    pad_src = jnp.zeros((pad,), i32)
    pad_dst = jnp.full((pad,), TRASH, i32)
    r2 = lambda a: a.reshape(ROWS_E, G)
    src_pre = r2(jnp.concatenate([edge_index_pre[0].astype(i32), pad_src]))
    dst_pre = r2(jnp.concatenate([edge_index_pre[1].astype(i32), pad_dst]))
    src_post = r2(jnp.concatenate([edge_index_post[0].astype(i32), pad_src]))
    dst_post = r2(jnp.concatenate([edge_index_post[1].astype(i32), pad_dst]))
    zz2 = jnp.zeros((ACC_ROWS, 16), f32)
    zz32 = jnp.zeros((ACC_ROWS, 32), f32)

    acc_pre, acc_post = _phase1(tab_pre, tab_post, src_pre, dst_pre,
                                src_post, dst_post, zz2)
    acc_pre = acc_pre[:, :N, 0:2]
    acc_post = acc_post[:, :N, 0:2]

    r = lambda b: b.reshape(1, -1)
    t1o, p1o = _phase2(acc_pre, acc_post, place_x, transition_x,
                       Wp, r(bp), Wt, r(bt),
                       Wl_pre_0, r(bl_pre_0), Wr_pre_0,
                       Wl_post_0, r(bl_post_0), Wr_post_0)

    Spre, Spost = _phase3(p1o.reshape(2 * N, 32), t1o.reshape(2 * N, 32),
                          src_pre, dst_pre, src_post, dst_post, zz32)
    Spre = Spre[:, :N, :]
    Spost = Spost[:, :N, :]

    return _phase4(Spre, Spost, acc_pre, acc_post, t1o, p1o,
                   Wl_pre_1, r(bl_pre_1), Wr_pre_1,
                   Wl_post_1, r(bl_post_1), Wr_post_1,
                   Wc1, r(bc1), Wc2, r(bc2))


# speedup vs baseline: 10.2489x; 1.0260x over previous
import functools

import jax
import jax.numpy as jnp
from jax import lax
from jax.experimental import pallas as pl
from jax.experimental.pallas import tpu as pltpu
from jax.experimental.pallas import tpu_sc as plsc

N = 50000
H = 64
E = 800000
NC = 2
NS = 16
G = 128
E_PAD = 802816
TRASH = N
ACC_ROWS = 50048
CHUNK = ACC_ROWS // NS
ROWS_E = E_PAD // G
G1 = ROWS_E // (NC * NS)
SEG1 = 28
CH1 = 14
G3 = ROWS_E // NS
SEG3 = 28
CH3 = 4
ZROWS = 136
BLK = 1000
NBLK = N // BLK

_mesh = plsc.VectorSubcoreMesh(core_axis_name="c", subcore_axis_name="s")


@functools.partial(
    pl.kernel,
    out_type=[jax.ShapeDtypeStruct((NC, ACC_ROWS, 16), jnp.float32),
              jax.ShapeDtypeStruct((NC, ACC_ROWS, 16), jnp.float32)],
    mesh=_mesh,
    scratch_types=[
        pltpu.VMEM((SEG1, G), jnp.int32),
        pltpu.VMEM((SEG1, G), jnp.int32),
        pltpu.VMEM((CH1, G, 16), jnp.float32),
        pltpu.VMEM((ZROWS, 16), jnp.float32),
        pltpu.VMEM_SHARED((ACC_ROWS, 16), jnp.float32),
        pltpu.SemaphoreType.DMA,
        pltpu.SemaphoreType.DMA,
    ],
    compiler_params=pltpu.CompilerParams(use_tc_tiling_on_sc=False),
)
def _phase1(tab_pre, tab_post, src_pre, dst_pre, src_post, dst_post,
            out_pre, out_post, sbuf, dbuf, rows, zbuf, acc, gsem, ssem):
    c = lax.axis_index("c")
    s = lax.axis_index("s")
    wid = c * NS + s

    def zfill(i, carry):
        zbuf[i] = jnp.zeros((16,), jnp.float32)
        return carry

    lax.fori_loop(0, ZROWS, zfill, 0)
    for tab, srcv, dstv, out in ((tab_pre, src_pre, dst_pre, out_pre),
                                 (tab_post, src_post, dst_post, out_post)):
        for k in range(CHUNK // ZROWS):
            pltpu.sync_copy(zbuf,
                            acc.at[pl.ds(s * CHUNK + k * ZROWS, ZROWS)])
        plsc.subcore_barrier()

        def seg_body(seg, carry, srcv=srcv, dstv=dstv, tab=tab):
            row0 = wid * G1 + seg * SEG1
            pltpu.sync_copy(srcv.at[pl.ds(row0, SEG1)], sbuf)
            pltpu.sync_copy(dstv.at[pl.ds(row0, SEG1)], dbuf)

            def ch_body(ck, carry2, tab=tab):
                gh = []
                for b in range(CH1):
                    r = ck * CH1 + b
                    gh.append(pltpu.async_copy(tab.at[sbuf.at[r]],
                                               rows.at[b], gsem))
                for h in gh:
                    h.wait()
                sh = []
                for b in range(CH1):
                    r = ck * CH1 + b
                    sh.append(pltpu.async_copy(rows.at[b], acc.at[dbuf.at[r]],
                                               ssem, add=True))
                for h in sh:
                    h.wait()
                return carry2

            lax.fori_loop(0, SEG1 // CH1, ch_body, 0)
            return carry

        lax.fori_loop(0, G1 // SEG1, seg_body, 0)
        plsc.subcore_barrier()
        pltpu.sync_copy(acc.at[pl.ds(s * CHUNK, CHUNK)],
                        out.at[c, pl.ds(s * CHUNK, CHUNK)])
        plsc.subcore_barrier()


@functools.partial(
    pl.kernel,
    out_type=[jax.ShapeDtypeStruct((NC, ACC_ROWS, 32), jnp.float32),
              jax.ShapeDtypeStruct((NC, ACC_ROWS, 32), jnp.float32)],
    mesh=_mesh,
    scratch_types=[
        pltpu.VMEM((SEG3, G), jnp.int32),
        pltpu.VMEM((SEG3, G), jnp.int32),
        pltpu.VMEM((ZROWS, 32), jnp.float32),
        pltpu.VMEM((CH3, G, 32), jnp.float32),
        pltpu.VMEM_SHARED((ACC_ROWS, 32), jnp.float32),
        pltpu.SemaphoreType.DMA,
        pltpu.SemaphoreType.DMA,
    ],
    compiler_params=pltpu.CompilerParams(use_tc_tiling_on_sc=False),
)
def _phase3(p1t, t1t, src_pre, dst_pre, src_post, dst_post,
            out_pre, out_post, sbuf, dbuf, zbuf, rows, acc, gsem, ssem):
    c = lax.axis_index("c")
    s = lax.axis_index("s")

    def zfill(i, carry):
        zbuf[i, pl.ds(0, 16)] = jnp.zeros((16,), jnp.float32)
        zbuf[i, pl.ds(16, 16)] = jnp.zeros((16,), jnp.float32)
        return carry

    lax.fori_loop(0, ZROWS, zfill, 0)
    for tab, srcv, dstv, out in ((p1t, src_pre, dst_pre, out_pre),
                                 (t1t, src_post, dst_post, out_post)):
        for k in range(CHUNK // ZROWS):
            pltpu.sync_copy(zbuf,
                            acc.at[pl.ds(s * CHUNK + k * ZROWS, ZROWS)])
        plsc.subcore_barrier()

        def seg_body(seg, carry, srcv=srcv, dstv=dstv, tab=tab):
            row0 = s * G3 + seg * SEG3
            pltpu.sync_copy(srcv.at[c, pl.ds(row0, SEG3)], sbuf)
            pltpu.sync_copy(dstv.at[pl.ds(row0, SEG3)], dbuf)

            def ch_body(ck, carry2, tab=tab):
                gh = []
                for b in range(CH3):
                    r = ck * CH3 + b
                    gh.append(pltpu.async_copy(tab.at[sbuf.at[r]],
                                               rows.at[b], gsem))
                for h in gh:
                    h.wait()
                sh = []
                for b in range(CH3):
                    r = ck * CH3 + b
                    sh.append(pltpu.async_copy(rows.at[b], acc.at[dbuf.at[r]],
                                               ssem, add=True))
                for h in sh:
                    h.wait()
                return carry2

            lax.fori_loop(0, SEG3 // CH3, ch_body, 0)
            return carry

        lax.fori_loop(0, G3 // SEG3, seg_body, 0)
        plsc.subcore_barrier()
        pltpu.sync_copy(acc.at[pl.ds(s * CHUNK, CHUNK)],
                        out.at[c, pl.ds(s * CHUNK, CHUNK)])
        plsc.subcore_barrier()


def _phase2_body(accpre_ref, accpost_ref, px_ref, tx_ref, Wp_ref, bp_ref,
                 Wt_ref, bt_ref, Wlp0_ref, blp0_ref, Wrp0_ref, Wlq0_ref,
                 blq0_ref, Wrq0_ref, t1o_ref, p1o_ref):
    f32 = jnp.float32
    accpre = accpre_ref[...]
    accpost = accpost_ref[...]
    spre = accpre[0, :, 0:1] + accpre[1, :, 0:1]
    cpre = accpre[0, :, 1:2] + accpre[1, :, 1:2]
    spost = accpost[0, :, 0:1] + accpost[1, :, 0:1]
    cpost = accpost[0, :, 1:2] + accpost[1, :, 1:2]
    upre = spre / jnp.maximum(cpre, 1.0)
    vpre = (cpre > 0).astype(f32)
    upost = spost / jnp.maximum(cpost, 1.0)
    vpost = (cpost > 0).astype(f32)
    Wp0 = Wp_ref[...]
    Wt0 = Wt_ref[...]
    Wlp0 = Wlp0_ref[...]
    Wrp0 = Wrp0_ref[...]
    Wlq0 = Wlq0_ref[...]
    Wrq0 = Wrq0_ref[...]
    bp = bp_ref[...]
    bt = bt_ref[...]
    dot = lambda a, b: jnp.dot(a, b, preferred_element_type=f32,
                               precision=jax.lax.Precision.HIGHEST)
    A1 = dot(Wp0, Wlp0)
    A2 = dot(bp, Wlp0)
    A3 = dot(Wt0, Wrp0)
    A4 = blp0_ref[...] + dot(bt, Wrp0)
    B1 = dot(Wt0, Wlq0)
    B2 = dot(bt, Wlq0)
    B3 = dot(Wp0, Wrq0)
    B4 = blq0_ref[...] + dot(bp, Wrq0)
    tx = tx_ref[...]
    px = px_ref[...]
    t1 = jnp.maximum(upre * A1 + vpre * A2 + tx * A3 + A4, 0.0)
    p1 = jnp.maximum(upost * B1 + vpost * B2 + px * B3 + B4, 0.0)
    t1o_ref[0] = t1[:, :32]
    t1o_ref[1] = t1[:, 32:]
    p1o_ref[0] = p1[:, :32]
    p1o_ref[1] = p1[:, 32:]


def _phase2(accpre, accpost, px, tx, Wp, bp, Wt, bt,
            Wlp0, blp0, Wrp0, Wlq0, blq0, Wrq0):
    full = lambda shape: pl.BlockSpec(shape, lambda i: tuple(0 for _ in shape))
    return pl.pallas_call(
        _phase2_body,
        grid=(NBLK,),
        in_specs=[
            pl.BlockSpec((2, BLK, 2), lambda i: (0, i, 0)),
            pl.BlockSpec((2, BLK, 2), lambda i: (0, i, 0)),
            pl.BlockSpec((BLK, 1), lambda i: (i, 0)),
            pl.BlockSpec((BLK, 1), lambda i: (i, 0)),
            full((1, H)), full((1, H)), full((1, H)), full((1, H)),
            full((H, H)), full((1, H)), full((H, H)),
            full((H, H)), full((1, H)), full((H, H)),
        ],
        out_specs=[
            pl.BlockSpec((2, BLK, 32), lambda i: (0, i, 0)),
            pl.BlockSpec((2, BLK, 32), lambda i: (0, i, 0)),
        ],
        out_shape=[jax.ShapeDtypeStruct((2, N, 32), jnp.float32),
                   jax.ShapeDtypeStruct((2, N, 32), jnp.float32)],
        compiler_params=pltpu.CompilerParams(
            dimension_semantics=("arbitrary",)),
    )(accpre, accpost, px, tx, Wp, bp, Wt, bt,
      Wlp0, blp0, Wrp0, Wlq0, blq0, Wrq0)


def _phase4_body(Spre_ref, Spost_ref, accpre_ref, accpost_ref, t1o_ref,
                 p1o_ref, Wlp1_ref, blp1_ref, Wrp1_ref, Wlq1_ref, blq1_ref,
                 Wrq1_ref, Wc1_ref, bc1_ref, Wc2_ref, bc2_ref, out_ref,
                 accP, accT):
    i = pl.program_id(0)
    f32 = jnp.float32
    dot = lambda a, b: jnp.dot(a, b, preferred_element_type=f32,
                               precision=jax.lax.Precision.HIGHEST)
    accpre = accpre_ref[...]
    accpost = accpost_ref[...]
    cpre = accpre[0, :, 1:2] + accpre[1, :, 1:2]
    cpost = accpost[0, :, 1:2] + accpost[1, :, 1:2]
    Spre = jnp.concatenate([Spre_ref[0], Spre_ref[1]], axis=1)
    Spost = jnp.concatenate([Spost_ref[0], Spost_ref[1]], axis=1)
    meanpre = Spre / jnp.maximum(cpre, 1.0)
    meanpost = Spost / jnp.maximum(cpost, 1.0)
    t1 = jnp.concatenate([t1o_ref[0], t1o_ref[1]], axis=1)
    p1 = jnp.concatenate([p1o_ref[0], p1o_ref[1]], axis=1)
    t2 = jnp.maximum(
        dot(meanpre, Wlp1_ref[...]) + blp1_ref[...] + dot(t1, Wrp1_ref[...]),
        0.0)
    p2 = jnp.maximum(
        dot(meanpost, Wlq1_ref[...]) + blq1_ref[...] + dot(p1, Wrq1_ref[...]),
        0.0)
    pt = jnp.sum(p2, axis=0, keepdims=True)
    tt = jnp.sum(t2, axis=0, keepdims=True)

    @pl.when(i == 0)
    def _():
        accP[...] = pt
        accT[...] = tt

    @pl.when(i > 0)
    def _():
        accP[...] += pt
        accT[...] += tt

    @pl.when(i == NBLK - 1)
    def _():
        mp = accP[...] / f32(N)
        mt = accT[...] / f32(N)
        g = jnp.concatenate([mp, mt], axis=1)
        h = jnp.maximum(dot(g, Wc1_ref[...]) + bc1_ref[...], 0.0)
        out_ref[...] = dot(h, Wc2_ref[...]) + bc2_ref[...]


def _phase4(Spre, Spost, accpre, accpost, t1o, p1o,
            Wlp1, blp1, Wrp1, Wlq1, blq1, Wrq1, Wc1, bc1, Wc2, bc2):
    full = lambda shape: pl.BlockSpec(shape, lambda i: tuple(0 for _ in shape))
    return pl.pallas_call(
        _phase4_body,
        grid=(NBLK,),
        in_specs=[
            pl.BlockSpec((2, BLK, 32), lambda i: (0, i, 0)),
            pl.BlockSpec((2, BLK, 32), lambda i: (0, i, 0)),
            pl.BlockSpec((2, BLK, 2), lambda i: (0, i, 0)),
            pl.BlockSpec((2, BLK, 2), lambda i: (0, i, 0)),
            pl.BlockSpec((2, BLK, 32), lambda i: (0, i, 0)),
            pl.BlockSpec((2, BLK, 32), lambda i: (0, i, 0)),
            full((H, H)), full((1, H)), full((H, H)),
            full((H, H)), full((1, H)), full((H, H)),
            full((2 * H, H)), full((1, H)), full((H, 2)), full((1, 2)),
        ],
        out_specs=[pl.BlockSpec((1, 2), lambda i: (0, 0))],
        out_shape=[jax.ShapeDtypeStruct((1, 2), jnp.float32)],
        scratch_shapes=[pltpu.VMEM((1, H), jnp.float32),
                        pltpu.VMEM((1, H), jnp.float32)],
        compiler_params=pltpu.CompilerParams(
            dimension_semantics=("arbitrary",)),
    )(Spre, Spost, accpre, accpost, t1o, p1o,
      Wlp1, blp1, Wrp1, Wlq1, blq1, Wrq1, Wc1, bc1, Wc2, bc2)[0]


def kernel(place_x, transition_x, edge_index_pre, edge_index_post, Wp, bp,
           Wt, bt, Wl_pre_0, bl_pre_0, Wr_pre_0, Wl_post_0, bl_post_0,
           Wr_post_0, Wl_pre_1, bl_pre_1, Wr_pre_1, Wl_post_1, bl_post_1,
           Wr_post_1, Wc1, bc1, Wc2, bc2):
    f32 = jnp.float32
    i32 = jnp.int32
    ones = jnp.ones_like(place_x)
    zpad = jnp.zeros((N, 14), f32)
    tab_pre = jnp.concatenate([place_x, ones, zpad], axis=1)
    tab_post = jnp.concatenate([transition_x, ones, zpad], axis=1)
    pad = E_PAD - E
    pad_src = jnp.zeros((pad,), i32)
    pad_dst = jnp.full((pad,), TRASH, i32)
    r2 = lambda a: a.reshape(ROWS_E, G)
    src_pre = r2(jnp.concatenate([edge_index_pre[0].astype(i32), pad_src]))
    dst_pre = r2(jnp.concatenate([edge_index_pre[1].astype(i32), pad_dst]))
    src_post = r2(jnp.concatenate([edge_index_post[0].astype(i32), pad_src]))
    dst_post = r2(jnp.concatenate([edge_index_post[1].astype(i32), pad_dst]))
    src_pre2 = jnp.stack([src_pre, src_pre + N])
    src_post2 = jnp.stack([src_post, src_post + N])

    acc_pre, acc_post = _phase1(tab_pre, tab_post, src_pre, dst_pre,
                                src_post, dst_post)
    acc_pre = acc_pre[:, :N, 0:2]
    acc_post = acc_post[:, :N, 0:2]

    r = lambda b: b.reshape(1, -1)
    t1o, p1o = _phase2(acc_pre, acc_post, place_x, transition_x,
                       Wp, r(bp), Wt, r(bt),
                       Wl_pre_0, r(bl_pre_0), Wr_pre_0,
                       Wl_post_0, r(bl_post_0), Wr_post_0)

    Spre, Spost = _phase3(p1o.reshape(2 * N, 32), t1o.reshape(2 * N, 32),
                          src_pre2, dst_pre, src_post2, dst_post)
    Spre = Spre[:, :N, :]
    Spost = Spost[:, :N, :]

    return _phase4(Spre, Spost, acc_pre, acc_post, t1o, p1o,
                   Wl_pre_1, r(bl_pre_1), Wr_pre_1,
                   Wl_post_1, r(bl_post_1), Wr_post_1,
                   Wc1, r(bc1), Wc2, r(bc2))


# speedup vs baseline: 11.2110x; 1.0939x over previous
import functools

import jax
import jax.numpy as jnp
from jax import lax
from jax.experimental import pallas as pl
from jax.experimental.pallas import tpu as pltpu
from jax.experimental.pallas import tpu_sc as plsc

N = 50000
H = 64
E = 800000
NC = 2
NS = 16
G = 128
E_PAD = 802816
TRASH = N
ACC_ROWS = 50048
CHUNK = ACC_ROWS // NS
ROWS_E = E_PAD // G
G1 = ROWS_E // (NC * NS)
SEG1 = 28
CH1 = 14
G3 = ROWS_E // NS
SEG3 = 28
CH3 = 4
ZROWS = 136
BLK = 1000
NBLK = N // BLK

_mesh = plsc.VectorSubcoreMesh(core_axis_name="c", subcore_axis_name="s")


@functools.partial(
    pl.kernel,
    out_type=[jax.ShapeDtypeStruct((NC, ACC_ROWS, 16), jnp.float32),
              jax.ShapeDtypeStruct((NC, ACC_ROWS, 16), jnp.float32)],
    mesh=_mesh,
    scratch_types=[
        pltpu.VMEM((SEG1, G), jnp.int32),
        pltpu.VMEM((SEG1, G), jnp.int32),
        pltpu.VMEM((CH1, G, 16), jnp.float32),
        pltpu.VMEM((ZROWS, 16), jnp.float32),
        pltpu.VMEM_SHARED((ACC_ROWS, 16), jnp.float32),
        pltpu.SemaphoreType.DMA,
        pltpu.SemaphoreType.DMA,
    ],
    compiler_params=pltpu.CompilerParams(use_tc_tiling_on_sc=False),
)
def _phase1(tab_pre, tab_post, src_pre, dst_pre, src_post, dst_post,
            out_pre, out_post, sbuf, dbuf, rows, zbuf, acc, gsem, ssem):
    c = lax.axis_index("c")
    s = lax.axis_index("s")
    wid = c * NS + s

    def zfill(i, carry):
        zbuf[i] = jnp.zeros((16,), jnp.float32)
        return carry

    lax.fori_loop(0, ZROWS, zfill, 0)
    for tab, srcv, dstv, out in ((tab_pre, src_pre, dst_pre, out_pre),
                                 (tab_post, src_post, dst_post, out_post)):
        for k in range(CHUNK // ZROWS):
            pltpu.sync_copy(zbuf,
                            acc.at[pl.ds(s * CHUNK + k * ZROWS, ZROWS)])
        plsc.subcore_barrier()

        def seg_body(seg, carry, srcv=srcv, dstv=dstv, tab=tab):
            row0 = wid * G1 + seg * SEG1
            pltpu.sync_copy(srcv.at[pl.ds(row0, SEG1)], sbuf)
            pltpu.sync_copy(dstv.at[pl.ds(row0, SEG1)], dbuf)

            def ch_body(ck, carry2, tab=tab):
                gh = []
                for b in range(CH1):
                    r = ck * CH1 + b
                    gh.append(pltpu.async_copy(tab.at[sbuf.at[r]],
                                               rows.at[b], gsem))
                for h in gh:
                    h.wait()
                sh = []
                for b in range(CH1):
                    r = ck * CH1 + b
                    sh.append(pltpu.async_copy(rows.at[b], acc.at[dbuf.at[r]],
                                               ssem, add=True))
                for h in sh:
                    h.wait()
                return carry2

            lax.fori_loop(0, SEG1 // CH1, ch_body, 0)
            return carry

        lax.fori_loop(0, G1 // SEG1, seg_body, 0)
        plsc.subcore_barrier()
        pltpu.sync_copy(acc.at[pl.ds(s * CHUNK, CHUNK)],
                        out.at[c, pl.ds(s * CHUNK, CHUNK)])
        plsc.subcore_barrier()


@functools.partial(
    pl.kernel,
    out_type=[jax.ShapeDtypeStruct((NC, ACC_ROWS, 32), jnp.float32),
              jax.ShapeDtypeStruct((NC, ACC_ROWS, 32), jnp.float32)],
    mesh=_mesh,
    scratch_types=[
        pltpu.VMEM((SEG3, G), jnp.int32),
        pltpu.VMEM((SEG3, G), jnp.int32),
        pltpu.VMEM((ZROWS, 32), jnp.float32),
        pltpu.VMEM((CH3, G, 32), jnp.float32),
        pltpu.VMEM_SHARED((ACC_ROWS, 32), jnp.float32),
        pltpu.SemaphoreType.DMA,
        pltpu.SemaphoreType.DMA,
    ],
    compiler_params=pltpu.CompilerParams(use_tc_tiling_on_sc=False),
)
def _phase3(p1t, t1t, src_pre, dst_pre, src_post, dst_post,
            out_pre, out_post, sbuf, dbuf, zbuf, rows, acc, gsem, ssem):
    c = lax.axis_index("c")
    s = lax.axis_index("s")

    def zfill(i, carry):
        zbuf[i, pl.ds(0, 16)] = jnp.zeros((16,), jnp.float32)
        zbuf[i, pl.ds(16, 16)] = jnp.zeros((16,), jnp.float32)
        return carry

    lax.fori_loop(0, ZROWS, zfill, 0)
    for tab, srcv, dstv, out in ((p1t, src_pre, dst_pre, out_pre),
                                 (t1t, src_post, dst_post, out_post)):
        for k in range(CHUNK // ZROWS):
            pltpu.sync_copy(zbuf,
                            acc.at[pl.ds(s * CHUNK + k * ZROWS, ZROWS)])
        plsc.subcore_barrier()

        def seg_body(seg, carry, srcv=srcv, dstv=dstv, tab=tab):
            row0 = s * G3 + seg * SEG3
            pltpu.sync_copy(srcv.at[c, pl.ds(row0, SEG3)], sbuf)
            pltpu.sync_copy(dstv.at[pl.ds(row0, SEG3)], dbuf)

            def ch_body(ck, carry2, tab=tab):
                gh = []
                for b in range(CH3):
                    r = ck * CH3 + b
                    gh.append(pltpu.async_copy(tab.at[sbuf.at[r]],
                                               rows.at[b], gsem))
                for h in gh:
                    h.wait()
                sh = []
                for b in range(CH3):
                    r = ck * CH3 + b
                    sh.append(pltpu.async_copy(rows.at[b], acc.at[dbuf.at[r]],
                                               ssem, add=True))
                for h in sh:
                    h.wait()
                return carry2

            lax.fori_loop(0, SEG3 // CH3, ch_body, 0)
            return carry

        lax.fori_loop(0, G3 // SEG3, seg_body, 0)
        plsc.subcore_barrier()
        pltpu.sync_copy(acc.at[pl.ds(s * CHUNK, CHUNK)],
                        out.at[c, pl.ds(s * CHUNK, CHUNK)])
        plsc.subcore_barrier()


def _phase2_body(accpre_ref, accpost_ref, px_ref, tx_ref, Wp_ref, bp_ref,
                 Wt_ref, bt_ref, Wlp0_ref, blp0_ref, Wrp0_ref, Wlq0_ref,
                 blq0_ref, Wrq0_ref, t1o_ref, p1o_ref):
    f32 = jnp.float32
    accpre = accpre_ref[...]
    accpost = accpost_ref[...]
    spre = accpre[0, :, 0:1] + accpre[1, :, 0:1]
    cpre = accpre[0, :, 1:2] + accpre[1, :, 1:2]
    spost = accpost[0, :, 0:1] + accpost[1, :, 0:1]
    cpost = accpost[0, :, 1:2] + accpost[1, :, 1:2]
    upre = spre / jnp.maximum(cpre, 1.0)
    vpre = (cpre > 0).astype(f32)
    upost = spost / jnp.maximum(cpost, 1.0)
    vpost = (cpost > 0).astype(f32)
    Wp0 = Wp_ref[...]
    Wt0 = Wt_ref[...]
    Wlp0 = Wlp0_ref[...]
    Wrp0 = Wrp0_ref[...]
    Wlq0 = Wlq0_ref[...]
    Wrq0 = Wrq0_ref[...]
    bp = bp_ref[...]
    bt = bt_ref[...]
    dot = lambda a, b: jnp.dot(a, b, preferred_element_type=f32,
                               precision=jax.lax.Precision.HIGHEST)
    A1 = dot(Wp0, Wlp0)
    A2 = dot(bp, Wlp0)
    A3 = dot(Wt0, Wrp0)
    A4 = blp0_ref[...] + dot(bt, Wrp0)
    B1 = dot(Wt0, Wlq0)
    B2 = dot(bt, Wlq0)
    B3 = dot(Wp0, Wrq0)
    B4 = blq0_ref[...] + dot(bp, Wrq0)
    tx = tx_ref[...]
    px = px_ref[...]
    t1 = jnp.maximum(upre * A1 + vpre * A2 + tx * A3 + A4, 0.0)
    p1 = jnp.maximum(upost * B1 + vpost * B2 + px * B3 + B4, 0.0)
    t1o_ref[0] = t1[:, :32]
    t1o_ref[1] = t1[:, 32:]
    p1o_ref[0] = p1[:, :32]
    p1o_ref[1] = p1[:, 32:]


def _phase2(accpre, accpost, px, tx, Wp, bp, Wt, bt,
            Wlp0, blp0, Wrp0, Wlq0, blq0, Wrq0):
    full = lambda shape: pl.BlockSpec(shape, lambda i: tuple(0 for _ in shape))
    return pl.pallas_call(
        _phase2_body,
        grid=(NBLK,),
        in_specs=[
            pl.BlockSpec((2, BLK, 16), lambda i: (0, i, 0)),
            pl.BlockSpec((2, BLK, 16), lambda i: (0, i, 0)),
            pl.BlockSpec((BLK, 1), lambda i: (i, 0)),
            pl.BlockSpec((BLK, 1), lambda i: (i, 0)),
            full((1, H)), full((1, H)), full((1, H)), full((1, H)),
            full((H, H)), full((1, H)), full((H, H)),
            full((H, H)), full((1, H)), full((H, H)),
        ],
        out_specs=[
            pl.BlockSpec((2, BLK, 32), lambda i: (0, i, 0)),
            pl.BlockSpec((2, BLK, 32), lambda i: (0, i, 0)),
        ],
        out_shape=[jax.ShapeDtypeStruct((2, N, 32), jnp.float32),
                   jax.ShapeDtypeStruct((2, N, 32), jnp.float32)],
        compiler_params=pltpu.CompilerParams(
            dimension_semantics=("arbitrary",)),
    )(accpre, accpost, px, tx, Wp, bp, Wt, bt,
      Wlp0, blp0, Wrp0, Wlq0, blq0, Wrq0)


def _phase4_body(Spre_ref, Spost_ref, accpre_ref, accpost_ref, t1o_ref,
                 p1o_ref, Wlp1_ref, blp1_ref, Wrp1_ref, Wlq1_ref, blq1_ref,
                 Wrq1_ref, Wc1_ref, bc1_ref, Wc2_ref, bc2_ref, out_ref,
                 accP, accT):
    i = pl.program_id(0)
    f32 = jnp.float32
    dot = lambda a, b: jnp.dot(a, b, preferred_element_type=f32,
                               precision=jax.lax.Precision.HIGHEST)
    accpre = accpre_ref[...]
    accpost = accpost_ref[...]
    cpre = accpre[0, :, 1:2] + accpre[1, :, 1:2]
    cpost = accpost[0, :, 1:2] + accpost[1, :, 1:2]
    Spre = jnp.concatenate([Spre_ref[0], Spre_ref[1]], axis=1)
    Spost = jnp.concatenate([Spost_ref[0], Spost_ref[1]], axis=1)
    meanpre = Spre / jnp.maximum(cpre, 1.0)
    meanpost = Spost / jnp.maximum(cpost, 1.0)
    t1 = jnp.concatenate([t1o_ref[0], t1o_ref[1]], axis=1)
    p1 = jnp.concatenate([p1o_ref[0], p1o_ref[1]], axis=1)
    t2 = jnp.maximum(
        dot(meanpre, Wlp1_ref[...]) + blp1_ref[...] + dot(t1, Wrp1_ref[...]),
        0.0)
    p2 = jnp.maximum(
        dot(meanpost, Wlq1_ref[...]) + blq1_ref[...] + dot(p1, Wrq1_ref[...]),
        0.0)
    pt = jnp.sum(p2, axis=0, keepdims=True)
    tt = jnp.sum(t2, axis=0, keepdims=True)

    @pl.when(i == 0)
    def _():
        accP[...] = pt
        accT[...] = tt

    @pl.when(i > 0)
    def _():
        accP[...] += pt
        accT[...] += tt

    @pl.when(i == NBLK - 1)
    def _():
        mp = accP[...] / f32(N)
        mt = accT[...] / f32(N)
        g = jnp.concatenate([mp, mt], axis=1)
        h = jnp.maximum(dot(g, Wc1_ref[...]) + bc1_ref[...], 0.0)
        out_ref[...] = dot(h, Wc2_ref[...]) + bc2_ref[...]


def _phase4(Spre, Spost, accpre, accpost, t1o, p1o,
            Wlp1, blp1, Wrp1, Wlq1, blq1, Wrq1, Wc1, bc1, Wc2, bc2):
    full = lambda shape: pl.BlockSpec(shape, lambda i: tuple(0 for _ in shape))
    return pl.pallas_call(
        _phase4_body,
        grid=(NBLK,),
        in_specs=[
            pl.BlockSpec((2, BLK, 32), lambda i: (0, i, 0)),
            pl.BlockSpec((2, BLK, 32), lambda i: (0, i, 0)),
            pl.BlockSpec((2, BLK, 16), lambda i: (0, i, 0)),
            pl.BlockSpec((2, BLK, 16), lambda i: (0, i, 0)),
            pl.BlockSpec((2, BLK, 32), lambda i: (0, i, 0)),
            pl.BlockSpec((2, BLK, 32), lambda i: (0, i, 0)),
            full((H, H)), full((1, H)), full((H, H)),
            full((H, H)), full((1, H)), full((H, H)),
            full((2 * H, H)), full((1, H)), full((H, 2)), full((1, 2)),
        ],
        out_specs=[pl.BlockSpec((1, 2), lambda i: (0, 0))],
        out_shape=[jax.ShapeDtypeStruct((1, 2), jnp.float32)],
        scratch_shapes=[pltpu.VMEM((1, H), jnp.float32),
                        pltpu.VMEM((1, H), jnp.float32)],
        compiler_params=pltpu.CompilerParams(
            dimension_semantics=("arbitrary",)),
    )(Spre, Spost, accpre, accpost, t1o, p1o,
      Wlp1, blp1, Wrp1, Wlq1, blq1, Wrq1, Wc1, bc1, Wc2, bc2)[0]


def kernel(place_x, transition_x, edge_index_pre, edge_index_post, Wp, bp,
           Wt, bt, Wl_pre_0, bl_pre_0, Wr_pre_0, Wl_post_0, bl_post_0,
           Wr_post_0, Wl_pre_1, bl_pre_1, Wr_pre_1, Wl_post_1, bl_post_1,
           Wr_post_1, Wc1, bc1, Wc2, bc2):
    f32 = jnp.float32
    i32 = jnp.int32
    ones = jnp.ones_like(place_x)
    zpad = jnp.zeros((N, 14), f32)
    tab_pre = jnp.concatenate([place_x, ones, zpad], axis=1)
    tab_post = jnp.concatenate([transition_x, ones, zpad], axis=1)
    pad = E_PAD - E
    pad_src = jnp.zeros((pad,), i32)
    pad_dst = jnp.full((pad,), TRASH, i32)
    r2 = lambda a: a.reshape(ROWS_E, G)
    src_pre = r2(jnp.concatenate([edge_index_pre[0].astype(i32), pad_src]))
    dst_pre = r2(jnp.concatenate([edge_index_pre[1].astype(i32), pad_dst]))
    src_post = r2(jnp.concatenate([edge_index_post[0].astype(i32), pad_src]))
    dst_post = r2(jnp.concatenate([edge_index_post[1].astype(i32), pad_dst]))
    src_pre2 = jnp.stack([src_pre, src_pre + N])
    src_post2 = jnp.stack([src_post, src_post + N])

    acc_pre, acc_post = _phase1(tab_pre, tab_post, src_pre, dst_pre,
                                src_post, dst_post)

    r = lambda b: b.reshape(1, -1)
    t1o, p1o = _phase2(acc_pre, acc_post, place_x, transition_x,
                       Wp, r(bp), Wt, r(bt),
                       Wl_pre_0, r(bl_pre_0), Wr_pre_0,
                       Wl_post_0, r(bl_post_0), Wr_post_0)

    Spre, Spost = _phase3(p1o.reshape(2 * N, 32), t1o.reshape(2 * N, 32),
                          src_pre2, dst_pre, src_post2, dst_post)

    return _phase4(Spre, Spost, acc_pre, acc_post, t1o, p1o,
                   Wl_pre_1, r(bl_pre_1), Wr_pre_1,
                   Wl_post_1, r(bl_post_1), Wr_post_1,
                   Wc1, r(bc1), Wc2, r(bc2))


# speedup vs baseline: 11.5898x; 1.0338x over previous
import functools

import jax
import jax.numpy as jnp
from jax import lax
from jax.experimental import pallas as pl
from jax.experimental.pallas import tpu as pltpu
from jax.experimental.pallas import tpu_sc as plsc

N = 50000
H = 64
E = 800000
NC = 2
NS = 16
G = 128
E_PAD = 802816
TRASH = N
ACC_ROWS = 50048
CHUNK = ACC_ROWS // NS
ROWS_E = E_PAD // G
G1 = ROWS_E // (NC * NS)
SEG1 = 28
CH1 = 14
G3 = ROWS_E // NS
SEG3 = 28
CH3 = 4
ZROWS = 136
BLK = 1000
NBLK = N // BLK

_mesh = plsc.VectorSubcoreMesh(core_axis_name="c", subcore_axis_name="s")


@functools.partial(
    pl.kernel,
    out_type=[jax.ShapeDtypeStruct((NC, ACC_ROWS, 16), jnp.float32),
              jax.ShapeDtypeStruct((NC, ACC_ROWS, 16), jnp.float32)],
    mesh=_mesh,
    scratch_types=[
        pltpu.VMEM((SEG1, G), jnp.int32),
        pltpu.VMEM((SEG1, G), jnp.int32),
        pltpu.VMEM((CH1, G, 16), jnp.float32),
        pltpu.VMEM((ZROWS, 16), jnp.float32),
        pltpu.VMEM_SHARED((ACC_ROWS, 16), jnp.float32),
        pltpu.SemaphoreType.DMA,
        pltpu.SemaphoreType.DMA,
    ],
    compiler_params=pltpu.CompilerParams(use_tc_tiling_on_sc=False),
)
def _phase1(tab_pre, tab_post, src_pre, dst_pre, src_post, dst_post,
            out_pre, out_post, sbuf, dbuf, rows, zbuf, acc, gsem, ssem):
    c = lax.axis_index("c")
    s = lax.axis_index("s")
    wid = c * NS + s

    def zfill(i, carry):
        zbuf[i] = jnp.zeros((16,), jnp.float32)
        return carry

    lax.fori_loop(0, ZROWS, zfill, 0)
    for tab, srcv, dstv, out in ((tab_pre, src_pre, dst_pre, out_pre),
                                 (tab_post, src_post, dst_post, out_post)):
        for k in range(CHUNK // ZROWS):
            pltpu.sync_copy(zbuf,
                            acc.at[pl.ds(s * CHUNK + k * ZROWS, ZROWS)])
        plsc.subcore_barrier()

        def seg_body(seg, carry, srcv=srcv, dstv=dstv, tab=tab):
            row0 = wid * G1 + seg * SEG1
            pltpu.sync_copy(srcv.at[pl.ds(row0, SEG1)], sbuf)
            pltpu.sync_copy(dstv.at[pl.ds(row0, SEG1)], dbuf)

            def ch_body(ck, carry2, tab=tab):
                gh = []
                for b in range(CH1):
                    r = ck * CH1 + b
                    gh.append(pltpu.async_copy(tab.at[sbuf.at[r]],
                                               rows.at[b], gsem))
                for h in gh:
                    h.wait()
                sh = []
                for b in range(CH1):
                    r = ck * CH1 + b
                    sh.append(pltpu.async_copy(rows.at[b], acc.at[dbuf.at[r]],
                                               ssem, add=True))
                for h in sh:
                    h.wait()
                return carry2

            lax.fori_loop(0, SEG1 // CH1, ch_body, 0)
            return carry

        lax.fori_loop(0, G1 // SEG1, seg_body, 0)
        plsc.subcore_barrier()
        pltpu.sync_copy(acc.at[pl.ds(s * CHUNK, CHUNK)],
                        out.at[c, pl.ds(s * CHUNK, CHUNK)])
        plsc.subcore_barrier()


@functools.partial(
    pl.kernel,
    out_type=[jax.ShapeDtypeStruct((NC, ACC_ROWS, 32), jnp.float32),
              jax.ShapeDtypeStruct((NC, ACC_ROWS, 32), jnp.float32)],
    mesh=_mesh,
    scratch_types=[
        pltpu.VMEM((SEG3, G), jnp.int32),
        pltpu.VMEM((SEG3, G), jnp.int32),
        pltpu.VMEM((ZROWS, 32), jnp.float32),
        pltpu.VMEM((CH3, G, 32), jnp.float32),
        pltpu.VMEM_SHARED((ACC_ROWS, 32), jnp.float32),
        pltpu.SemaphoreType.DMA,
        pltpu.SemaphoreType.DMA,
    ],
    compiler_params=pltpu.CompilerParams(use_tc_tiling_on_sc=False),
)
def _phase3(p1t, t1t, src_pre, dst_pre, src_post, dst_post,
            out_pre, out_post, sbuf, dbuf, zbuf, rows, acc, gsem, ssem):
    c = lax.axis_index("c")
    s = lax.axis_index("s")

    def zfill(i, carry):
        zbuf[i, pl.ds(0, 16)] = jnp.zeros((16,), jnp.float32)
        zbuf[i, pl.ds(16, 16)] = jnp.zeros((16,), jnp.float32)
        return carry

    lax.fori_loop(0, ZROWS, zfill, 0)
    for tab, srcv, dstv, out in ((p1t, src_pre, dst_pre, out_pre),
                                 (t1t, src_post, dst_post, out_post)):
        for k in range(CHUNK // ZROWS):
            pltpu.sync_copy(zbuf,
                            acc.at[pl.ds(s * CHUNK + k * ZROWS, ZROWS)])
        plsc.subcore_barrier()

        def seg_body(seg, carry, srcv=srcv, dstv=dstv, tab=tab):
            row0 = s * G3 + seg * SEG3
            pltpu.sync_copy(srcv.at[c, pl.ds(row0, SEG3)], sbuf)
            pltpu.sync_copy(dstv.at[pl.ds(row0, SEG3)], dbuf)

            def ch_body(ck, carry2, tab=tab):
                gh = []
                for b in range(CH3):
                    r = ck * CH3 + b
                    gh.append(pltpu.async_copy(tab.at[sbuf.at[r]],
                                               rows.at[b], gsem))
                for h in gh:
                    h.wait()
                sh = []
                for b in range(CH3):
                    r = ck * CH3 + b
                    sh.append(pltpu.async_copy(rows.at[b], acc.at[dbuf.at[r]],
                                               ssem, add=True))
                for h in sh:
                    h.wait()
                return carry2

            lax.fori_loop(0, SEG3 // CH3, ch_body, 0)
            return carry

        lax.fori_loop(0, G3 // SEG3, seg_body, 0)
        plsc.subcore_barrier()
        pltpu.sync_copy(acc.at[pl.ds(s * CHUNK, CHUNK)],
                        out.at[c, pl.ds(s * CHUNK, CHUNK)])
        plsc.subcore_barrier()


def _phase0_body(Wp_ref, bp_ref, Wt_ref, bt_ref, Wlp0_ref, blp0_ref,
                 Wrp0_ref, Wlq0_ref, blq0_ref, Wrq0_ref, ab_ref):
    f32 = jnp.float32
    dot = lambda a, b: jnp.dot(a, b, preferred_element_type=f32,
                               precision=jax.lax.Precision.HIGHEST)
    Wp0 = Wp_ref[...]
    Wt0 = Wt_ref[...]
    bp = bp_ref[...]
    bt = bt_ref[...]
    ab_ref[0:1] = dot(Wp0, Wlp0_ref[...])
    ab_ref[1:2] = dot(bp, Wlp0_ref[...])
    ab_ref[2:3] = dot(Wt0, Wrp0_ref[...])
    ab_ref[3:4] = blp0_ref[...] + dot(bt, Wrp0_ref[...])
    ab_ref[4:5] = dot(Wt0, Wlq0_ref[...])
    ab_ref[5:6] = dot(bt, Wlq0_ref[...])
    ab_ref[6:7] = dot(Wp0, Wrq0_ref[...])
    ab_ref[7:8] = blq0_ref[...] + dot(bp, Wrq0_ref[...])


def _phase0(Wp, bp, Wt, bt, Wlp0, blp0, Wrp0, Wlq0, blq0, Wrq0):
    full = lambda shape: pl.BlockSpec(shape, lambda: tuple(0 for _ in shape))
    return pl.pallas_call(
        _phase0_body,
        in_specs=[
            full((1, H)), full((1, H)), full((1, H)), full((1, H)),
            full((H, H)), full((1, H)), full((H, H)),
            full((H, H)), full((1, H)), full((H, H)),
        ],
        out_specs=full((8, H)),
        out_shape=jax.ShapeDtypeStruct((8, H), jnp.float32),
    )(Wp, bp, Wt, bt, Wlp0, blp0, Wrp0, Wlq0, blq0, Wrq0)


def _phase2_body(accpre_ref, accpost_ref, px_ref, tx_ref, ab_ref,
                 t1o_ref, p1o_ref):
    f32 = jnp.float32
    accpre = accpre_ref[...]
    accpost = accpost_ref[...]
    spre = accpre[0, :, 0:1] + accpre[1, :, 0:1]
    cpre = accpre[0, :, 1:2] + accpre[1, :, 1:2]
    spost = accpost[0, :, 0:1] + accpost[1, :, 0:1]
    cpost = accpost[0, :, 1:2] + accpost[1, :, 1:2]
    upre = spre / jnp.maximum(cpre, 1.0)
    vpre = (cpre > 0).astype(f32)
    upost = spost / jnp.maximum(cpost, 1.0)
    vpost = (cpost > 0).astype(f32)
    ab = ab_ref[...]
    tx = tx_ref[...]
    px = px_ref[...]
    t1 = jnp.maximum(
        upre * ab[0:1] + vpre * ab[1:2] + tx * ab[2:3] + ab[3:4], 0.0)
    p1 = jnp.maximum(
        upost * ab[4:5] + vpost * ab[5:6] + px * ab[6:7] + ab[7:8], 0.0)
    t1o_ref[0] = t1[:, :32]
    t1o_ref[1] = t1[:, 32:]
    p1o_ref[0] = p1[:, :32]
    p1o_ref[1] = p1[:, 32:]


def _phase2(accpre, accpost, px, tx, ab):
    full = lambda shape: pl.BlockSpec(shape, lambda i: tuple(0 for _ in shape))
    return pl.pallas_call(
        _phase2_body,
        grid=(NBLK,),
        in_specs=[
            pl.BlockSpec((2, BLK, 16), lambda i: (0, i, 0)),
            pl.BlockSpec((2, BLK, 16), lambda i: (0, i, 0)),
            pl.BlockSpec((BLK, 1), lambda i: (i, 0)),
            pl.BlockSpec((BLK, 1), lambda i: (i, 0)),
            full((8, H)),
        ],
        out_specs=[
            pl.BlockSpec((2, BLK, 32), lambda i: (0, i, 0)),
            pl.BlockSpec((2, BLK, 32), lambda i: (0, i, 0)),
        ],
        out_shape=[jax.ShapeDtypeStruct((2, N, 32), jnp.float32),
                   jax.ShapeDtypeStruct((2, N, 32), jnp.float32)],
        compiler_params=pltpu.CompilerParams(
            dimension_semantics=("arbitrary",)),
    )(accpre, accpost, px, tx, ab)


def _phase4_body(Spre_ref, Spost_ref, accpre_ref, accpost_ref, t1o_ref,
                 p1o_ref, Wlp1_ref, blp1_ref, Wrp1_ref, Wlq1_ref, blq1_ref,
                 Wrq1_ref, Wc1_ref, bc1_ref, Wc2_ref, bc2_ref, out_ref,
                 accP, accT):
    i = pl.program_id(0)
    f32 = jnp.float32
    dot = lambda a, b: jnp.dot(a, b, preferred_element_type=f32,
                               precision=jax.lax.Precision.HIGHEST)
    accpre = accpre_ref[...]
    accpost = accpost_ref[...]
    cpre = accpre[0, :, 1:2] + accpre[1, :, 1:2]
    cpost = accpost[0, :, 1:2] + accpost[1, :, 1:2]
    Spre = jnp.concatenate([Spre_ref[0], Spre_ref[1]], axis=1)
    Spost = jnp.concatenate([Spost_ref[0], Spost_ref[1]], axis=1)
    meanpre = Spre / jnp.maximum(cpre, 1.0)
    meanpost = Spost / jnp.maximum(cpost, 1.0)
    t1 = jnp.concatenate([t1o_ref[0], t1o_ref[1]], axis=1)
    p1 = jnp.concatenate([p1o_ref[0], p1o_ref[1]], axis=1)
    t2 = jnp.maximum(
        dot(meanpre, Wlp1_ref[...]) + blp1_ref[...] + dot(t1, Wrp1_ref[...]),
        0.0)
    p2 = jnp.maximum(
        dot(meanpost, Wlq1_ref[...]) + blq1_ref[...] + dot(p1, Wrq1_ref[...]),
        0.0)
    pt = jnp.sum(p2, axis=0, keepdims=True)
    tt = jnp.sum(t2, axis=0, keepdims=True)

    @pl.when(i == 0)
    def _():
        accP[...] = pt
        accT[...] = tt

    @pl.when(i > 0)
    def _():
        accP[...] += pt
        accT[...] += tt

    @pl.when(i == NBLK - 1)
    def _():
        mp = accP[...] / f32(N)
        mt = accT[...] / f32(N)
        g = jnp.concatenate([mp, mt], axis=1)
        h = jnp.maximum(dot(g, Wc1_ref[...]) + bc1_ref[...], 0.0)
        out_ref[...] = dot(h, Wc2_ref[...]) + bc2_ref[...]


def _phase4(Spre, Spost, accpre, accpost, t1o, p1o,
            Wlp1, blp1, Wrp1, Wlq1, blq1, Wrq1, Wc1, bc1, Wc2, bc2):
    full = lambda shape: pl.BlockSpec(shape, lambda i: tuple(0 for _ in shape))
    return pl.pallas_call(
        _phase4_body,
        grid=(NBLK,),
        in_specs=[
            pl.BlockSpec((2, BLK, 32), lambda i: (0, i, 0)),
            pl.BlockSpec((2, BLK, 32), lambda i: (0, i, 0)),
            pl.BlockSpec((2, BLK, 16), lambda i: (0, i, 0)),
            pl.BlockSpec((2, BLK, 16), lambda i: (0, i, 0)),
            pl.BlockSpec((2, BLK, 32), lambda i: (0, i, 0)),
            pl.BlockSpec((2, BLK, 32), lambda i: (0, i, 0)),
            full((H, H)), full((1, H)), full((H, H)),
            full((H, H)), full((1, H)), full((H, H)),
            full((2 * H, H)), full((1, H)), full((H, 2)), full((1, 2)),
        ],
        out_specs=[pl.BlockSpec((1, 2), lambda i: (0, 0))],
        out_shape=[jax.ShapeDtypeStruct((1, 2), jnp.float32)],
        scratch_shapes=[pltpu.VMEM((1, H), jnp.float32),
                        pltpu.VMEM((1, H), jnp.float32)],
        compiler_params=pltpu.CompilerParams(
            dimension_semantics=("arbitrary",)),
    )(Spre, Spost, accpre, accpost, t1o, p1o,
      Wlp1, blp1, Wrp1, Wlq1, blq1, Wrq1, Wc1, bc1, Wc2, bc2)[0]


def kernel(place_x, transition_x, edge_index_pre, edge_index_post, Wp, bp,
           Wt, bt, Wl_pre_0, bl_pre_0, Wr_pre_0, Wl_post_0, bl_post_0,
           Wr_post_0, Wl_pre_1, bl_pre_1, Wr_pre_1, Wl_post_1, bl_post_1,
           Wr_post_1, Wc1, bc1, Wc2, bc2):
    f32 = jnp.float32
    i32 = jnp.int32
    ones = jnp.ones_like(place_x)
    zpad = jnp.zeros((N, 14), f32)
    tab_pre = jnp.concatenate([place_x, ones, zpad], axis=1)
    tab_post = jnp.concatenate([transition_x, ones, zpad], axis=1)
    pad = E_PAD - E
    pad_blk = jnp.stack([jnp.zeros((pad,), i32), jnp.full((pad,), TRASH, i32)])
    ei_pre = jnp.concatenate([edge_index_pre.astype(i32), pad_blk],
                             axis=1).reshape(2, ROWS_E, G)
    ei_post = jnp.concatenate([edge_index_post.astype(i32), pad_blk],
                              axis=1).reshape(2, ROWS_E, G)
    src_pre = ei_pre[0]
    dst_pre = ei_pre[1]
    src_post = ei_post[0]
    dst_post = ei_post[1]
    src_pre2 = jnp.stack([src_pre, src_pre + N])
    src_post2 = jnp.stack([src_post, src_post + N])

    r = lambda b: b.reshape(1, -1)
    ab = _phase0(Wp, r(bp), Wt, r(bt),
                 Wl_pre_0, r(bl_pre_0), Wr_pre_0,
                 Wl_post_0, r(bl_post_0), Wr_post_0)

    acc_pre, acc_post = _phase1(tab_pre, tab_post, src_pre, dst_pre,
                                src_post, dst_post)

    t1o, p1o = _phase2(acc_pre, acc_post, place_x, transition_x, ab)

    Spre, Spost = _phase3(p1o.reshape(2 * N, 32), t1o.reshape(2 * N, 32),
                          src_pre2, dst_pre, src_post2, dst_post)

    return _phase4(Spre, Spost, acc_pre, acc_post, t1o, p1o,
                   Wl_pre_1, r(bl_pre_1), Wr_pre_1,
                   Wl_post_1, r(bl_post_1), Wr_post_1,
                   Wc1, r(bc1), Wc2, r(bc2))


# speedup vs baseline: 12.4331x; 1.0728x over previous
import functools

import jax
import jax.numpy as jnp
from jax import lax
from jax.experimental import pallas as pl
from jax.experimental.pallas import tpu as pltpu
from jax.experimental.pallas import tpu_sc as plsc

N = 50000
H = 64
E = 800000
NC = 2
NS = 16
G = 128
E_PAD = 802816
TRASH = N
ACC_ROWS = 50048
CHUNK = ACC_ROWS // NS
ROWS_E = E_PAD // G
G1 = ROWS_E // (NC * NS)
SEG1 = 28
CH1 = 14
G3 = ROWS_E // NS
SEG3 = 28
CH3 = 4
ZROWS = 136
BLK = 2000
NBLK = N // BLK

_mesh = plsc.VectorSubcoreMesh(core_axis_name="c", subcore_axis_name="s")


@functools.partial(
    pl.kernel,
    out_type=[jax.ShapeDtypeStruct((NC, ACC_ROWS, 16), jnp.float32),
              jax.ShapeDtypeStruct((NC, ACC_ROWS, 16), jnp.float32)],
    mesh=_mesh,
    scratch_types=[
        pltpu.VMEM((SEG1, G), jnp.int32),
        pltpu.VMEM((SEG1, G), jnp.int32),
        pltpu.VMEM((CH1, G, 16), jnp.float32),
        pltpu.VMEM((ZROWS, 16), jnp.float32),
        pltpu.VMEM_SHARED((ACC_ROWS, 16), jnp.float32),
        pltpu.SemaphoreType.DMA,
        pltpu.SemaphoreType.DMA,
    ],
    compiler_params=pltpu.CompilerParams(use_tc_tiling_on_sc=False),
)
def _phase1(tab_pre, tab_post, src_pre, dst_pre, src_post, dst_post,
            out_pre, out_post, sbuf, dbuf, rows, zbuf, acc, gsem, ssem):
    c = lax.axis_index("c")
    s = lax.axis_index("s")
    wid = c * NS + s

    def zfill(i, carry):
        zbuf[i] = jnp.zeros((16,), jnp.float32)
        return carry

    lax.fori_loop(0, ZROWS, zfill, 0)
    for tab, srcv, dstv, out in ((tab_pre, src_pre, dst_pre, out_pre),
                                 (tab_post, src_post, dst_post, out_post)):
        for k in range(CHUNK // ZROWS):
            pltpu.sync_copy(zbuf,
                            acc.at[pl.ds(s * CHUNK + k * ZROWS, ZROWS)])
        plsc.subcore_barrier()

        def seg_body(seg, carry, srcv=srcv, dstv=dstv, tab=tab):
            row0 = wid * G1 + seg * SEG1
            pltpu.sync_copy(srcv.at[pl.ds(row0, SEG1)], sbuf)
            pltpu.sync_copy(dstv.at[pl.ds(row0, SEG1)], dbuf)

            def ch_body(ck, carry2, tab=tab):
                gh = []
                for b in range(CH1):
                    r = ck * CH1 + b
                    gh.append(pltpu.async_copy(tab.at[sbuf.at[r]],
                                               rows.at[b], gsem))
                for h in gh:
                    h.wait()
                sh = []
                for b in range(CH1):
                    r = ck * CH1 + b
                    sh.append(pltpu.async_copy(rows.at[b], acc.at[dbuf.at[r]],
                                               ssem, add=True))
                for h in sh:
                    h.wait()
                return carry2

            lax.fori_loop(0, SEG1 // CH1, ch_body, 0)
            return carry

        lax.fori_loop(0, G1 // SEG1, seg_body, 0)
        plsc.subcore_barrier()
        pltpu.sync_copy(acc.at[pl.ds(s * CHUNK, CHUNK)],
                        out.at[c, pl.ds(s * CHUNK, CHUNK)])
        plsc.subcore_barrier()


@functools.partial(
    pl.kernel,
    out_type=[jax.ShapeDtypeStruct((NC, ACC_ROWS, 32), jnp.float32),
              jax.ShapeDtypeStruct((NC, ACC_ROWS, 32), jnp.float32)],
    mesh=_mesh,
    scratch_types=[
        pltpu.VMEM((SEG3, G), jnp.int32),
        pltpu.VMEM((SEG3, G), jnp.int32),
        pltpu.VMEM((ZROWS, 32), jnp.float32),
        pltpu.VMEM((CH3, G, 32), jnp.float32),
        pltpu.VMEM_SHARED((ACC_ROWS, 32), jnp.float32),
        pltpu.SemaphoreType.DMA,
        pltpu.SemaphoreType.DMA,
    ],
    compiler_params=pltpu.CompilerParams(use_tc_tiling_on_sc=False),
)
def _phase3(p1t, t1t, src_pre, dst_pre, src_post, dst_post,
            out_pre, out_post, sbuf, dbuf, zbuf, rows, acc, gsem, ssem):
    c = lax.axis_index("c")
    s = lax.axis_index("s")

    def zfill(i, carry):
        zbuf[i, pl.ds(0, 16)] = jnp.zeros((16,), jnp.float32)
        zbuf[i, pl.ds(16, 16)] = jnp.zeros((16,), jnp.float32)
        return carry

    lax.fori_loop(0, ZROWS, zfill, 0)
    for tab, srcv, dstv, out in ((p1t, src_pre, dst_pre, out_pre),
                                 (t1t, src_post, dst_post, out_post)):
        for k in range(CHUNK // ZROWS):
            pltpu.sync_copy(zbuf,
                            acc.at[pl.ds(s * CHUNK + k * ZROWS, ZROWS)])
        plsc.subcore_barrier()

        def seg_body(seg, carry, srcv=srcv, dstv=dstv, tab=tab):
            row0 = s * G3 + seg * SEG3
            pltpu.sync_copy(srcv.at[c, pl.ds(row0, SEG3)], sbuf)
            pltpu.sync_copy(dstv.at[pl.ds(row0, SEG3)], dbuf)

            def ch_body(ck, carry2, tab=tab):
                gh = []
                for b in range(CH3):
                    r = ck * CH3 + b
                    gh.append(pltpu.async_copy(tab.at[sbuf.at[r]],
                                               rows.at[b], gsem))
                for h in gh:
                    h.wait()
                sh = []
                for b in range(CH3):
                    r = ck * CH3 + b
                    sh.append(pltpu.async_copy(rows.at[b], acc.at[dbuf.at[r]],
                                               ssem, add=True))
                for h in sh:
                    h.wait()
                return carry2

            lax.fori_loop(0, SEG3 // CH3, ch_body, 0)
            return carry

        lax.fori_loop(0, G3 // SEG3, seg_body, 0)
        plsc.subcore_barrier()
        pltpu.sync_copy(acc.at[pl.ds(s * CHUNK, CHUNK)],
                        out.at[c, pl.ds(s * CHUNK, CHUNK)])
        plsc.subcore_barrier()


def _phase0_body(Wp_ref, bp_ref, Wt_ref, bt_ref, Wlp0_ref, blp0_ref,
                 Wrp0_ref, Wlq0_ref, blq0_ref, Wrq0_ref, ab_ref):
    f32 = jnp.float32
    dot = lambda a, b: jnp.dot(a, b, preferred_element_type=f32,
                               precision=jax.lax.Precision.HIGHEST)
    Wp0 = Wp_ref[...]
    Wt0 = Wt_ref[...]
    bp = bp_ref[...]
    bt = bt_ref[...]
    ab_ref[0:1] = dot(Wp0, Wlp0_ref[...])
    ab_ref[1:2] = dot(bp, Wlp0_ref[...])
    ab_ref[2:3] = dot(Wt0, Wrp0_ref[...])
    ab_ref[3:4] = blp0_ref[...] + dot(bt, Wrp0_ref[...])
    ab_ref[4:5] = dot(Wt0, Wlq0_ref[...])
    ab_ref[5:6] = dot(bt, Wlq0_ref[...])
    ab_ref[6:7] = dot(Wp0, Wrq0_ref[...])
    ab_ref[7:8] = blq0_ref[...] + dot(bp, Wrq0_ref[...])


def _phase0(Wp, bp, Wt, bt, Wlp0, blp0, Wrp0, Wlq0, blq0, Wrq0):
    full = lambda shape: pl.BlockSpec(shape, lambda: tuple(0 for _ in shape))
    return pl.pallas_call(
        _phase0_body,
        in_specs=[
            full((1, H)), full((1, H)), full((1, H)), full((1, H)),
            full((H, H)), full((1, H)), full((H, H)),
            full((H, H)), full((1, H)), full((H, H)),
        ],
        out_specs=full((8, H)),
        out_shape=jax.ShapeDtypeStruct((8, H), jnp.float32),
    )(Wp, bp, Wt, bt, Wlp0, blp0, Wrp0, Wlq0, blq0, Wrq0)


def _phase2_body(accpre_ref, accpost_ref, px_ref, tx_ref, ab_ref,
                 t1o_ref, p1o_ref):
    f32 = jnp.float32
    accpre = accpre_ref[...]
    accpost = accpost_ref[...]
    spre = accpre[0, :, 0:1] + accpre[1, :, 0:1]
    cpre = accpre[0, :, 1:2] + accpre[1, :, 1:2]
    spost = accpost[0, :, 0:1] + accpost[1, :, 0:1]
    cpost = accpost[0, :, 1:2] + accpost[1, :, 1:2]
    upre = spre / jnp.maximum(cpre, 1.0)
    vpre = (cpre > 0).astype(f32)
    upost = spost / jnp.maximum(cpost, 1.0)
    vpost = (cpost > 0).astype(f32)
    ab = ab_ref[...]
    tx = tx_ref[...]
    px = px_ref[...]
    t1 = jnp.maximum(
        upre * ab[0:1] + vpre * ab[1:2] + tx * ab[2:3] + ab[3:4], 0.0)
    p1 = jnp.maximum(
        upost * ab[4:5] + vpost * ab[5:6] + px * ab[6:7] + ab[7:8], 0.0)
    t1o_ref[0] = t1[:, :32]
    t1o_ref[1] = t1[:, 32:]
    p1o_ref[0] = p1[:, :32]
    p1o_ref[1] = p1[:, 32:]


def _phase2(accpre, accpost, px, tx, ab):
    full = lambda shape: pl.BlockSpec(shape, lambda i: tuple(0 for _ in shape))
    return pl.pallas_call(
        _phase2_body,
        grid=(NBLK,),
        in_specs=[
            pl.BlockSpec((2, BLK, 16), lambda i: (0, i, 0)),
            pl.BlockSpec((2, BLK, 16), lambda i: (0, i, 0)),
            pl.BlockSpec((BLK, 1), lambda i: (i, 0)),
            pl.BlockSpec((BLK, 1), lambda i: (i, 0)),
            full((8, H)),
        ],
        out_specs=[
            pl.BlockSpec((2, BLK, 32), lambda i: (0, i, 0)),
            pl.BlockSpec((2, BLK, 32), lambda i: (0, i, 0)),
        ],
        out_shape=[jax.ShapeDtypeStruct((2, N, 32), jnp.float32),
                   jax.ShapeDtypeStruct((2, N, 32), jnp.float32)],
        compiler_params=pltpu.CompilerParams(
            dimension_semantics=("arbitrary",)),
    )(accpre, accpost, px, tx, ab)


def _phase4_body(Spre_ref, Spost_ref, accpre_ref, accpost_ref, t1o_ref,
                 p1o_ref, Wlp1_ref, blp1_ref, Wrp1_ref, Wlq1_ref, blq1_ref,
                 Wrq1_ref, Wc1_ref, bc1_ref, Wc2_ref, bc2_ref, out_ref,
                 accP, accT):
    i = pl.program_id(0)
    f32 = jnp.float32
    dot = lambda a, b: jnp.dot(a, b, preferred_element_type=f32,
                               precision=jax.lax.Precision.HIGHEST)
    accpre = accpre_ref[...]
    accpost = accpost_ref[...]
    cpre = accpre[0, :, 1:2] + accpre[1, :, 1:2]
    cpost = accpost[0, :, 1:2] + accpost[1, :, 1:2]
    Spre = jnp.concatenate([Spre_ref[0], Spre_ref[1]], axis=1)
    Spost = jnp.concatenate([Spost_ref[0], Spost_ref[1]], axis=1)
    meanpre = Spre / jnp.maximum(cpre, 1.0)
    meanpost = Spost / jnp.maximum(cpost, 1.0)
    t1 = jnp.concatenate([t1o_ref[0], t1o_ref[1]], axis=1)
    p1 = jnp.concatenate([p1o_ref[0], p1o_ref[1]], axis=1)
    t2 = jnp.maximum(
        dot(meanpre, Wlp1_ref[...]) + blp1_ref[...] + dot(t1, Wrp1_ref[...]),
        0.0)
    p2 = jnp.maximum(
        dot(meanpost, Wlq1_ref[...]) + blq1_ref[...] + dot(p1, Wrq1_ref[...]),
        0.0)
    pt = jnp.sum(p2, axis=0, keepdims=True)
    tt = jnp.sum(t2, axis=0, keepdims=True)

    @pl.when(i == 0)
    def _():
        accP[...] = pt
        accT[...] = tt

    @pl.when(i > 0)
    def _():
        accP[...] += pt
        accT[...] += tt

    @pl.when(i == NBLK - 1)
    def _():
        mp = accP[...] / f32(N)
        mt = accT[...] / f32(N)
        g = jnp.concatenate([mp, mt], axis=1)
        h = jnp.maximum(dot(g, Wc1_ref[...]) + bc1_ref[...], 0.0)
        out_ref[...] = dot(h, Wc2_ref[...]) + bc2_ref[...]


def _phase4(Spre, Spost, accpre, accpost, t1o, p1o,
            Wlp1, blp1, Wrp1, Wlq1, blq1, Wrq1, Wc1, bc1, Wc2, bc2):
    full = lambda shape: pl.BlockSpec(shape, lambda i: tuple(0 for _ in shape))
    return pl.pallas_call(
        _phase4_body,
        grid=(NBLK,),
        in_specs=[
            pl.BlockSpec((2, BLK, 32), lambda i: (0, i, 0)),
            pl.BlockSpec((2, BLK, 32), lambda i: (0, i, 0)),
            pl.BlockSpec((2, BLK, 16), lambda i: (0, i, 0)),
            pl.BlockSpec((2, BLK, 16), lambda i: (0, i, 0)),
            pl.BlockSpec((2, BLK, 32), lambda i: (0, i, 0)),
            pl.BlockSpec((2, BLK, 32), lambda i: (0, i, 0)),
            full((H, H)), full((1, H)), full((H, H)),
            full((H, H)), full((1, H)), full((H, H)),
            full((2 * H, H)), full((1, H)), full((H, 2)), full((1, 2)),
        ],
        out_specs=[pl.BlockSpec((1, 2), lambda i: (0, 0))],
        out_shape=[jax.ShapeDtypeStruct((1, 2), jnp.float32)],
        scratch_shapes=[pltpu.VMEM((1, H), jnp.float32),
                        pltpu.VMEM((1, H), jnp.float32)],
        compiler_params=pltpu.CompilerParams(
            dimension_semantics=("arbitrary",)),
    )(Spre, Spost, accpre, accpost, t1o, p1o,
      Wlp1, blp1, Wrp1, Wlq1, blq1, Wrq1, Wc1, bc1, Wc2, bc2)[0]


def kernel(place_x, transition_x, edge_index_pre, edge_index_post, Wp, bp,
           Wt, bt, Wl_pre_0, bl_pre_0, Wr_pre_0, Wl_post_0, bl_post_0,
           Wr_post_0, Wl_pre_1, bl_pre_1, Wr_pre_1, Wl_post_1, bl_post_1,
           Wr_post_1, Wc1, bc1, Wc2, bc2):
    f32 = jnp.float32
    i32 = jnp.int32
    ones = jnp.ones_like(place_x)
    zpad = jnp.zeros((N, 14), f32)
    tab_pre = jnp.concatenate([place_x, ones, zpad], axis=1)
    tab_post = jnp.concatenate([transition_x, ones, zpad], axis=1)
    pad = E_PAD - E
    pad_blk = jnp.stack([jnp.zeros((pad,), i32), jnp.full((pad,), TRASH, i32)])
    ei_pre = jnp.concatenate([edge_index_pre.astype(i32), pad_blk],
                             axis=1).reshape(2, ROWS_E, G)
    ei_post = jnp.concatenate([edge_index_post.astype(i32), pad_blk],
                              axis=1).reshape(2, ROWS_E, G)
    src_pre = ei_pre[0]
    dst_pre = ei_pre[1]
    src_post = ei_post[0]
    dst_post = ei_post[1]
    src_pre2 = jnp.stack([src_pre, src_pre + N])
    src_post2 = jnp.stack([src_post, src_post + N])

    r = lambda b: b.reshape(1, -1)
    ab = _phase0(Wp, r(bp), Wt, r(bt),
                 Wl_pre_0, r(bl_pre_0), Wr_pre_0,
                 Wl_post_0, r(bl_post_0), Wr_post_0)

    acc_pre, acc_post = _phase1(tab_pre, tab_post, src_pre, dst_pre,
                                src_post, dst_post)

    t1o, p1o = _phase2(acc_pre, acc_post, place_x, transition_x, ab)

    Spre, Spost = _phase3(p1o.reshape(2 * N, 32), t1o.reshape(2 * N, 32),
                          src_pre2, dst_pre, src_post2, dst_post)

    return _phase4(Spre, Spost, acc_pre, acc_post, t1o, p1o,
                   Wl_pre_1, r(bl_pre_1), Wr_pre_1,
                   Wl_post_1, r(bl_post_1), Wr_post_1,
                   Wc1, r(bc1), Wc2, r(bc2))
